# Initial kernel scaffold; baseline (speedup 1.0000x reference)
import jax
import jax.numpy as jnp
from jax.experimental import pallas as pl


def kernel(node_features, edge_index, previous_embedding, params):
    raise NotImplementedError("write your pallas kernel here")


# speedup vs baseline: 78.7025x; 78.7025x over previous
import functools

import jax
import jax.numpy as jnp
from jax import lax
from jax.experimental import pallas as pl
from jax.experimental.pallas import tpu as pltpu
from jax.experimental.pallas import tpu_sc as plsc

N = 10000
E = 320000
D = 128
H = 8
DH = 16
TOPO = 64

XW = 144
BLK = 2000
GRID = N // BLK
CHUNK = 128
NCHUNK = E // CHUNK
NWORK = 32
KMAX = -(-NCHUNK // NWORK)
NP = 10240
RPT = NP // 16
WSL = 128


def _seg_mats():
    k = lax.broadcasted_iota(jnp.int32, (D, H), 0) // DH
    h = lax.broadcasted_iota(jnp.int32, (D, H), 1)
    S = (k == h).astype(jnp.float32)
    return S, S.T


def _alpha_tail(xp, asf, adf):
    S, _ = _seg_mats()
    a_s = jnp.dot(xp * asf, S, preferred_element_type=jnp.float32)
    a_d = jnp.dot(xp * adf, S, preferred_element_type=jnp.float32)
    xpa = jnp.concatenate([xp, a_s, jnp.zeros_like(a_s)], axis=-1)
    adt = jnp.concatenate([a_d, a_d], axis=-1)
    return xpa, adt


def _proj_body(nf_ref, win_ref, bin_ref, w_ref, b_ref, asf_ref, adf_ref,
               x_ref, xpa_ref, adt_ref):
    x = jnp.dot(nf_ref[...], win_ref[...], preferred_element_type=jnp.float32)
    x = jnp.maximum(x + bin_ref[...], 0.0)
    xp = jnp.dot(x, w_ref[...], preferred_element_type=jnp.float32) + b_ref[...]
    x_ref[...] = x
    xpa_ref[...], adt_ref[...] = _alpha_tail(xp, asf_ref[...], adf_ref[...])


def _combine_body(acc_ref, x_ref, w_ref, b_ref, asf_ref, adf_ref,
                  xn_ref, xpa_ref, adt_ref):
    _, ST = _seg_mats()
    acc = acc_ref[0] + acc_ref[1]
    msg = acc[:, :D]
    den = acc[:, D:D + H] + 1e-9
    xn = jnp.maximum(msg / jnp.dot(den, ST, preferred_element_type=jnp.float32)
                     + x_ref[...], 0.0)
    xp = jnp.dot(xn, w_ref[...], preferred_element_type=jnp.float32) + b_ref[...]
    xn_ref[...] = xn
    xpa_ref[...], adt_ref[...] = _alpha_tail(xp, asf_ref[...], adf_ref[...])


def _final_body(acc_ref, x_ref, w_ref, b_ref, sum_ref):
    i = pl.program_id(0)
    _, ST = _seg_mats()
    acc = acc_ref[0] + acc_ref[1]
    msg = acc[:, :D]
    den = acc[:, D:D + H] + 1e-9
    xn = jnp.maximum(msg / jnp.dot(den, ST, preferred_element_type=jnp.float32)
                     + x_ref[...], 0.0)
    y = jnp.dot(xn, w_ref[...], preferred_element_type=jnp.float32) + b_ref[...]
    part = jnp.sum(y, axis=0, keepdims=True)

    @pl.when(i == 0)
    def _():
        sum_ref[...] = part

    @pl.when(i > 0)
    def _():
        sum_ref[...] = sum_ref[...] + part


def _head_body(sum_ref, prev_ref, wt1_ref, bt1_ref, wt2r_ref, bt2_ref,
               wi_ref, wh_ref, bi_ref, bh_ref,
               cur_ref, mem_ref, cp_ref):
    cur = sum_ref[...] * (1.0 / N)
    prev = prev_ref[...]
    diff = cur - prev
    h1 = jnp.tanh(jnp.dot(diff, wt1_ref[...], preferred_element_type=jnp.float32)
                  + bt1_ref[...])
    logit = jnp.sum(h1 * wt2r_ref[...], axis=-1, keepdims=True) + bt2_ref[...]
    cp_ref[...] = jax.nn.sigmoid(logit)

    def gru(h, xt):
        gi = jnp.dot(xt, wi_ref[...], preferred_element_type=jnp.float32) + bi_ref[...]
        gh = jnp.dot(h, wh_ref[...], preferred_element_type=jnp.float32) + bh_ref[...]
        r = jax.nn.sigmoid(gi[:, :D] + gh[:, :D])
        z = jax.nn.sigmoid(gi[:, D:2 * D] + gh[:, D:2 * D])
        n = jnp.tanh(gi[:, 2 * D:] + r * gh[:, 2 * D:])
        return (1.0 - z) * n + z * h

    h = gru(jnp.zeros((1, D), jnp.float32), prev)
    h = gru(h, cur)
    cur_ref[...] = cur
    mem_ref[...] = h


_f32 = jnp.float32


def _row_spec(w):
    return pl.BlockSpec((BLK, w), lambda i: (i, 0))


def _full_spec(shape):
    return pl.BlockSpec(shape, lambda i: tuple(0 for _ in shape))


_proj = pl.pallas_call(
    _proj_body,
    grid=(GRID,),
    in_specs=[_row_spec(D), _full_spec((D, D)), _full_spec((1, D)),
              _full_spec((D, D)), _full_spec((1, D)),
              _full_spec((1, D)), _full_spec((1, D))],
    out_specs=[_row_spec(D), _row_spec(XW), _row_spec(2 * H)],
    out_shape=[jax.ShapeDtypeStruct((N, D), _f32),
               jax.ShapeDtypeStruct((N, XW), _f32),
               jax.ShapeDtypeStruct((N, 2 * H), _f32)],
)

_combine = pl.pallas_call(
    _combine_body,
    grid=(GRID,),
    in_specs=[pl.BlockSpec((2, BLK, XW), lambda i: (0, i, 0)), _row_spec(D),
              _full_spec((D, D)), _full_spec((1, D)),
              _full_spec((1, D)), _full_spec((1, D))],
    out_specs=[_row_spec(D), _row_spec(XW), _row_spec(2 * H)],
    out_shape=[jax.ShapeDtypeStruct((N, D), _f32),
               jax.ShapeDtypeStruct((N, XW), _f32),
               jax.ShapeDtypeStruct((N, 2 * H), _f32)],
)

_final = pl.pallas_call(
    _final_body,
    grid=(GRID,),
    in_specs=[pl.BlockSpec((2, BLK, XW), lambda i: (0, i, 0)), _row_spec(D),
              _full_spec((D, D)), _full_spec((1, D))],
    out_specs=pl.BlockSpec((1, D), lambda i: (0, 0)),
    out_shape=jax.ShapeDtypeStruct((1, D), _f32),
)

_head = pl.pallas_call(
    _head_body,
    grid=(1,),
    in_specs=[_full_spec((1, D)), _full_spec((1, D)),
              _full_spec((D, TOPO)), _full_spec((1, TOPO)),
              _full_spec((1, TOPO)), _full_spec((1, 1)),
              _full_spec((D, 3 * D)), _full_spec((D, 3 * D)),
              _full_spec((1, 3 * D)), _full_spec((1, 3 * D))],
    out_specs=[_full_spec((1, D)), _full_spec((1, D)), _full_spec((1, 1))],
    out_shape=[jax.ShapeDtypeStruct((1, D), _f32),
               jax.ShapeDtypeStruct((1, D), _f32),
               jax.ShapeDtypeStruct((1, 1), _f32)],
)


def _vgather(v, idx):
    dn = lax.GatherDimensionNumbers(offset_dims=(), collapsed_slice_dims=(0,),
                                    start_index_map=(0,))
    return lax.gather(v, idx[:, None], dn, (1,),
                      mode=lax.GatherScatterMode.PROMISE_IN_BOUNDS)


def _sc_edge_body(xpa_hbm, adt_hbm, src_hbm, dst_hbm, acc_out,
                  acc_sh, bufx, bufad, sidx, didx,
                  semx, sema):
    c = lax.axis_index("c")
    s = lax.axis_index("s")
    wid = s * 2 + c
    base = s * RPT

    def zbody(r, _):
        for j in range(XW // 16):
            bufx[r, pl.ds(j * 16, 16)] = jnp.zeros((16,), jnp.float32)
        return 0
    lax.fori_loop(0, WSL, zbody, 0)

    for j in range(5):
        pltpu.sync_copy(bufx, acc_sh.at[pl.ds(base + j * WSL, WSL)])
    plsc.subcore_barrier()

    def chunk(k, _):
        cid = k * NWORK + wid

        @pl.when(cid < NCHUNK)
        def _():
            ebase = cid * CHUNK
            pltpu.sync_copy(src_hbm.at[pl.ds(ebase, CHUNK)], sidx)
            pltpu.sync_copy(dst_hbm.at[pl.ds(ebase, CHUNK)], didx)
            cpx = pltpu.async_copy(xpa_hbm.at[sidx], bufx, semx)
            cpa = pltpu.async_copy(adt_hbm.at[didx], bufad, sema)
            cpa.wait()
            cpx.wait()

            def ebody(e, _):
                dp = pl.ds(D, 16)
                v = bufx[e, dp] + bufad[e, pl.ds(0, 16)]
                v = jnp.where(v >= 0.0, v, 0.2 * v)
                pv = jnp.exp(v)
                bufx[e, dp] = pv
                for h in range(H):
                    sc = _vgather(pv, jnp.full((16,), h, jnp.int32))
                    d0 = pl.ds(h * DH, DH)
                    bufx[e, d0] = bufx[e, d0] * sc
                return 0
            lax.fori_loop(0, CHUNK, ebody, 0)

            pltpu.sync_copy(bufx, acc_sh.at[didx], add=True)
        return 0
    lax.fori_loop(0, KMAX, chunk, 0)
    plsc.subcore_barrier()

    for j in range(5):
        pltpu.sync_copy(acc_sh.at[pl.ds(base + j * WSL, WSL)], bufx)
        pltpu.sync_copy(bufx, acc_out.at[c, pl.ds(base + j * WSL, WSL)])


@functools.cache
def _sc_edge_call():
    return pl.kernel(
        _sc_edge_body,
        out_type=jax.ShapeDtypeStruct((2, NP, XW), _f32),
        mesh=plsc.VectorSubcoreMesh(core_axis_name="c", subcore_axis_name="s"),
        compiler_params=pltpu.CompilerParams(use_tc_tiling_on_sc=False),
        scratch_types=[
            pltpu.VMEM_SHARED((NP, XW), _f32),
            pltpu.VMEM((CHUNK, XW), _f32),
            pltpu.VMEM((CHUNK, 2 * H), _f32),
            pltpu.VMEM((CHUNK,), jnp.int32),
            pltpu.VMEM((CHUNK,), jnp.int32),
            pltpu.SemaphoreType.DMA,
            pltpu.SemaphoreType.DMA,
        ],
    )


def kernel(node_features, edge_index, previous_embedding, params):
    p = params
    src = edge_index[0]
    dst = edge_index[1]

    def row(v):
        return v.reshape(1, -1)

    x, xpa, adt = _proj(node_features, p['W_in'], row(p['b_in']),
                        p['W_0'], row(p['b_0']),
                        row(p['asrc_0']), row(p['adst_0']))
    for l in range(3):
        acc = _sc_edge_call()(xpa, adt, src, dst)
        if l < 2:
            x, xpa, adt = _combine(acc, x, p[f'W_{l + 1}'], row(p[f'b_{l + 1}']),
                                   row(p[f'asrc_{l + 1}']), row(p[f'adst_{l + 1}']))
        else:
            colsum = _final(acc, x, p['W_out'], row(p['b_out']))

    cur, mem, cp = _head(colsum, row(previous_embedding),
                         p['Wt1'], row(p['bt1']), p['Wt2'].reshape(1, TOPO),
                         p['bt2'].reshape(1, 1),
                         p['Wi'], p['Wh'], row(p['bi']), row(p['bh']))
    return cur.reshape(D), mem.reshape(D), cp.reshape(1)
